# Initial kernel scaffold; baseline (speedup 1.0000x reference)
import jax
import jax.numpy as jnp
from jax.experimental import pallas as pl


def kernel(pieces, ranks, files, tiles, factorization_mask, values, lengths):
    raise NotImplementedError("write your pallas kernel here")


# speedup vs baseline: 452.6382x; 452.6382x over previous
import functools

import jax
import jax.numpy as jnp
from jax import lax
from jax.experimental import pallas as pl
from jax.experimental.pallas import tpu as pltpu
from jax.experimental.pallas import tpu_sc as plsc

K = 12
DOUT = 256
B = 16384
N = B * 32
NV = K * 64
NC, NS = 2, 16
NW = NC * NS
BPW = B // NW
CBAGS = 128
NCHUNK = BPW // CBAGS
VSTAGE = CBAGS * 32 + 16
TSTAGE = 2048
PAD = 4096


def _build_w2(pieces, ranks, files, tiles, mask):
    def body(p_ref, r_ref, f_ref, t_ref, m_ref, o_ref):
        merged = t_ref[...] + (p_ref[...] + r_ref[...] + f_ref[...]) * m_ref[...]
        o_ref[:, :DOUT] = merged.reshape(NV, DOUT)
        rolled = jnp.concatenate([merged[K // 2:], merged[:K // 2]], axis=0)
        flipped = jnp.concatenate(
            [rolled[:, 7 - i:8 - i] for i in range(8)], axis=1)
        o_ref[:, DOUT:] = flipped.reshape(NV, DOUT)

    return pl.pallas_call(
        body,
        out_shape=jax.ShapeDtypeStruct((NV, 2 * DOUT), jnp.float32),
    )(pieces, ranks, files, tiles, mask)


def _sc_hist(values_padded, lengths):
    mesh = plsc.VectorSubcoreMesh(core_axis_name="c", subcore_axis_name="s")

    @functools.partial(
        pl.kernel,
        out_type=[
            jax.ShapeDtypeStruct((B * NV,), jnp.float32),
            jax.ShapeDtypeStruct((NW * NV,), jnp.float32),
        ],
        mesh=mesh,
        compiler_params=pltpu.CompilerParams(needs_layout_passes=False),
        scratch_types=[
            pltpu.VMEM((B + 16,), jnp.int32),
            pltpu.VMEM((VSTAGE,), jnp.int32),
            pltpu.VMEM((CBAGS * NV,), jnp.float32),
            pltpu.VMEM((NV,), jnp.float32),
            pltpu.VMEM((TSTAGE + 16,), jnp.int32),
        ],
    )
    def k(values_hbm, lengths_hbm, hist_hbm, tail_hbm,
          len_v, vals_v, hist_v, tacc_v, tstage_v):
        wid = lax.axis_index("s") * NC + lax.axis_index("c")
        lane = lax.iota(jnp.int32, 16)
        ones = jnp.ones((16,), jnp.float32)
        zeros16 = jnp.zeros((16,), jnp.float32)

        pltpu.sync_copy(lengths_hbm, len_v.at[pl.ds(0, B)])

        my_first = wid * BPW

        def pre_body(q, carry):
            tot_vec, base_vec = carry
            chunk = len_v[pl.ds(q * 16, 16)]
            inc = (q * 16 < my_first).astype(jnp.int32)
            return tot_vec + chunk, base_vec + chunk * inc

        zi = jnp.zeros((16,), jnp.int32)
        tot_vec, base_vec = lax.fori_loop(0, B // 16, pre_body, (zi, zi))

        def _hsum(v):
            s = v[0]
            for i in range(1, 16):
                s = s + v[i]
            return s

        total = _hsum(tot_vec)
        base = _hsum(base_vec)

        def chunk_body(c, off):
            def zbody(z, _):
                for u in range(16):
                    hist_v[pl.ds(z * 256 + u * 16, 16)] = zeros16
                return 0
            lax.fori_loop(0, CBAGS * NV // 256, zbody, 0)

            dma_base = (off // 8) * 8
            pltpu.sync_copy(values_hbm.at[pl.ds(dma_base, VSTAGE)], vals_v)

            def bag_body(i, o):
                ln = len_v[pl.ds(my_first + c * CBAGS + i, 16)][0]
                lo = o - dma_base
                v0 = plsc.load_gather(vals_v, [lo + lane])
                m0 = (lane < ln) & (v0 < NV)
                plsc.addupdate_scatter(hist_v, [i * NV + v0], ones, mask=m0)

                @pl.when(ln > 16)
                def _():
                    v1 = plsc.load_gather(vals_v, [lo + 16 + lane])
                    m1 = (lane + 16 < ln) & (v1 < NV)
                    plsc.addupdate_scatter(hist_v, [i * NV + v1], ones, mask=m1)

                return o + ln

            off = lax.fori_loop(0, CBAGS, bag_body, off)
            pltpu.sync_copy(
                hist_v,
                hist_hbm.at[pl.ds((my_first + c * CBAGS) * NV, CBAGS * NV)])
            return off

        lax.fori_loop(0, NCHUNK, chunk_body, base)

        def tz(z, _):
            tacc_v[pl.ds(z * 16, 16)] = zeros16
            return 0
        lax.fori_loop(0, NV // 16, tz, 0)

        n_tail = N - total
        gpw = (n_tail + 16 * NW - 1) // (16 * NW)
        start_w = total + wid * gpw * 16
        span = gpw * 16
        limit = jnp.minimum(jnp.int32(N), start_w + span)
        n_out = (span + TSTAGE - 1) // TSTAGE

        def touter(o, _):
            st = start_w + o * TSTAGE
            st_al = (st // 8) * 8
            pltpu.sync_copy(values_hbm.at[pl.ds(st_al, TSTAGE + 16)], tstage_v)
            shift = st - st_al

            def tinner(g, _):
                pos = st + g * 16
                vv = plsc.load_gather(tstage_v, [shift + g * 16 + lane])
                mm = ((pos + lane) < limit) & (vv < NV)
                plsc.addupdate_scatter(tacc_v, [vv], ones, mask=mm)
                return 0

            lax.fori_loop(0, TSTAGE // 16, tinner, 0)
            return 0

        lax.fori_loop(0, n_out, touter, 0)
        pltpu.sync_copy(tacc_v, tail_hbm.at[pl.ds(wid * NV, NV)])

    return k(values_padded, lengths)


def _matmul(hist, w2, tail):
    BM = 1024
    nb = B // BM

    def body(h_ref, w2_ref, t_ref, o_ref):
        acc = jnp.dot(h_ref[...], w2_ref[...],
                      preferred_element_type=jnp.float32)
        tvec = jnp.sum(t_ref[...], axis=0, keepdims=True)
        tcon = jnp.dot(tvec, w2_ref[...],
                       preferred_element_type=jnp.float32)
        row = lax.broadcasted_iota(jnp.int32, (BM, 1), 0)
        sel = (row == BM - 1) & (pl.program_id(0) == nb - 1)
        acc = acc + jnp.where(sel, tcon, 0.0)
        o_ref[...] = jnp.clip(acc, 0.0, 1.0)

    return pl.pallas_call(
        body,
        grid=(nb,),
        in_specs=[
            pl.BlockSpec((BM, NV), lambda i: (i, 0)),
            pl.BlockSpec((NV, 2 * DOUT), lambda i: (0, 0)),
            pl.BlockSpec((NW, NV), lambda i: (0, 0)),
        ],
        out_specs=pl.BlockSpec((BM, 2 * DOUT), lambda i: (i, 0)),
        out_shape=jax.ShapeDtypeStruct((B, 2 * DOUT), jnp.float32),
    )(hist, w2, tail)


def kernel(pieces, ranks, files, tiles, factorization_mask, values, lengths):
    w2 = _build_w2(pieces, ranks, files, tiles, factorization_mask)
    values_padded = jnp.pad(values, (0, PAD))
    hist_flat, tail_flat = _sc_hist(values_padded, lengths)
    hist = hist_flat.reshape(B, NV)
    tail = tail_flat.reshape(NW, NV)
    out = _matmul(hist, w2, tail)
    return out[:, :DOUT], out[:, DOUT:]


# speedup vs baseline: 695.4656x; 1.5365x over previous
import functools

import jax
import jax.numpy as jnp
from jax import lax
from jax.experimental import pallas as pl
from jax.experimental.pallas import tpu as pltpu
from jax.experimental.pallas import tpu_sc as plsc

K = 12
DOUT = 256
B = 16384
N = B * 32
NV = K * 64
NC, NS = 2, 16
NW = NC * NS
BPW = B // NW
CBAGS = 128
NCHUNK = BPW // CBAGS
VSTAGE = CBAGS * 32 + 16
TSTAGE = 2048
PAD = 4096


def _build_w2(pieces, ranks, files, tiles, mask):
    def body(p_ref, r_ref, f_ref, t_ref, m_ref, o_ref):
        merged = t_ref[...] + (p_ref[...] + r_ref[...] + f_ref[...]) * m_ref[...]
        o_ref[:, :DOUT] = merged.reshape(NV, DOUT)
        rolled = jnp.concatenate([merged[K // 2:], merged[:K // 2]], axis=0)
        flipped = jnp.concatenate(
            [rolled[:, 7 - i:8 - i] for i in range(8)], axis=1)
        o_ref[:, DOUT:] = flipped.reshape(NV, DOUT)

    return pl.pallas_call(
        body,
        out_shape=jax.ShapeDtypeStruct((NV, 2 * DOUT), jnp.float32),
    )(pieces, ranks, files, tiles, mask)


def _sc_hist(values_padded, lengths):
    mesh = plsc.VectorSubcoreMesh(core_axis_name="c", subcore_axis_name="s")

    @functools.partial(
        pl.kernel,
        out_type=[
            jax.ShapeDtypeStruct((B * NV,), jnp.float32),
            jax.ShapeDtypeStruct((NW * NV,), jnp.float32),
        ],
        mesh=mesh,
        compiler_params=pltpu.CompilerParams(needs_layout_passes=False),
        scratch_types=[
            pltpu.VMEM((B + 16,), jnp.int32),
            pltpu.VMEM((VSTAGE,), jnp.int32),
            pltpu.VMEM((CBAGS * NV,), jnp.float32),
            pltpu.VMEM((NV,), jnp.float32),
            pltpu.VMEM((TSTAGE + 16,), jnp.int32),
        ],
    )
    def k(values_hbm, lengths_hbm, hist_hbm, tail_hbm,
          len_v, vals_v, hist_v, tacc_v, tstage_v):
        wid = lax.axis_index("s") * NC + lax.axis_index("c")
        lane = lax.iota(jnp.int32, 16)
        ones = jnp.ones((16,), jnp.float32)
        zeros16 = jnp.zeros((16,), jnp.float32)

        pltpu.sync_copy(lengths_hbm, len_v.at[pl.ds(0, B)])

        my_first = wid * BPW

        def pre_body(q, carry):
            tot_vec, base_vec = carry
            chunk = len_v[pl.ds(q * 16, 16)]
            inc = (q * 16 < my_first).astype(jnp.int32)
            return tot_vec + chunk, base_vec + chunk * inc

        zi = jnp.zeros((16,), jnp.int32)
        tot_vec, base_vec = lax.fori_loop(0, B // 16, pre_body, (zi, zi))

        def _hsum(v):
            s = v[0]
            for i in range(1, 16):
                s = s + v[i]
            return s

        total = _hsum(tot_vec)
        base = _hsum(base_vec)

        def chunk_body(c, off):
            def zbody(z, _):
                for u in range(16):
                    hist_v[pl.ds(z * 256 + u * 16, 16)] = zeros16
                return 0
            lax.fori_loop(0, CBAGS * NV // 256, zbody, 0)

            dma_base = (off // 8) * 8
            pltpu.sync_copy(values_hbm.at[pl.ds(dma_base, VSTAGE)], vals_v)

            def bag_body(i, o):
                ln = len_v[pl.ds(my_first + c * CBAGS + i, 16)][0]
                lo = o - dma_base
                ibase = i * 128

                def scat(v, m):
                    idx = ((v >> 7) * (CBAGS * 128)) + ibase + (v & 127)
                    plsc.addupdate_scatter(hist_v, [idx], ones, mask=m)

                v0 = plsc.load_gather(vals_v, [lo + lane])
                scat(v0, (lane < ln) & (v0 < NV))

                @pl.when(ln > 16)
                def _():
                    v1 = plsc.load_gather(vals_v, [lo + 16 + lane])
                    scat(v1, (lane + 16 < ln) & (v1 < NV))

                return o + ln

            off = lax.fori_loop(0, CBAGS, bag_body, off)
            row0 = my_first + c * CBAGS
            for vt in range(NV // 128):
                pltpu.sync_copy(
                    hist_v.at[pl.ds(vt * CBAGS * 128, CBAGS * 128)],
                    hist_hbm.at[pl.ds(vt * B * 128 + row0 * 128, CBAGS * 128)])
            return off

        lax.fori_loop(0, NCHUNK, chunk_body, base)

        def tz(z, _):
            tacc_v[pl.ds(z * 16, 16)] = zeros16
            return 0
        lax.fori_loop(0, NV // 16, tz, 0)

        n_tail = N - total
        gpw = (n_tail + 16 * NW - 1) // (16 * NW)
        start_w = total + wid * gpw * 16
        span = gpw * 16
        limit = jnp.minimum(jnp.int32(N), start_w + span)
        n_out = (span + TSTAGE - 1) // TSTAGE

        def touter(o, _):
            st = start_w + o * TSTAGE
            st_al = (st // 8) * 8
            pltpu.sync_copy(values_hbm.at[pl.ds(st_al, TSTAGE + 16)], tstage_v)
            shift = st - st_al

            def tinner(g, _):
                pos = st + g * 16
                vv = plsc.load_gather(tstage_v, [shift + g * 16 + lane])
                mm = ((pos + lane) < limit) & (vv < NV)
                plsc.addupdate_scatter(tacc_v, [vv], ones, mask=mm)
                return 0

            lax.fori_loop(0, TSTAGE // 16, tinner, 0)
            return 0

        lax.fori_loop(0, n_out, touter, 0)
        pltpu.sync_copy(tacc_v, tail_hbm.at[pl.ds(wid * NV, NV)])

    return k(values_padded, lengths)


def _matmul(hist_flat, w2, tail):
    BM = 1024
    nb = B // BM

    def body(h_ref, w2_ref, t_ref, oa_ref, ob_ref):
        acc = jnp.dot(h_ref[0], w2_ref[:128, :],
                      preferred_element_type=jnp.float32)
        for j in range(1, NV // 128):
            acc = acc + jnp.dot(h_ref[j], w2_ref[j * 128:(j + 1) * 128, :],
                                preferred_element_type=jnp.float32)
        tvec = jnp.sum(t_ref[...], axis=0, keepdims=True)
        tcon = jnp.dot(tvec, w2_ref[...],
                       preferred_element_type=jnp.float32)
        row = lax.broadcasted_iota(jnp.int32, (BM, 1), 0)
        sel = (row == BM - 1) & (pl.program_id(0) == nb - 1)
        acc = jnp.clip(acc + jnp.where(sel, tcon, 0.0), 0.0, 1.0)
        oa_ref[...] = acc[:, :DOUT]
        ob_ref[...] = acc[:, DOUT:]

    return pl.pallas_call(
        body,
        grid=(nb,),
        in_specs=[
            pl.BlockSpec((NV // 128, BM, 128), lambda i: (0, i, 0)),
            pl.BlockSpec((NV, 2 * DOUT), lambda i: (0, 0)),
            pl.BlockSpec((NW, NV), lambda i: (0, 0)),
        ],
        out_specs=[
            pl.BlockSpec((BM, DOUT), lambda i: (i, 0)),
            pl.BlockSpec((BM, DOUT), lambda i: (i, 0)),
        ],
        out_shape=[
            jax.ShapeDtypeStruct((B, DOUT), jnp.float32),
            jax.ShapeDtypeStruct((B, DOUT), jnp.float32),
        ],
        compiler_params=pltpu.CompilerParams(
            dimension_semantics=("parallel",)),
    )(hist_flat, w2, tail)


def kernel(pieces, ranks, files, tiles, factorization_mask, values, lengths):
    w2 = _build_w2(pieces, ranks, files, tiles, factorization_mask)
    values_padded = jnp.pad(values, (0, PAD))
    hist_flat, tail_flat = _sc_hist(values_padded, lengths)
    hist3 = hist_flat.reshape(NV // 128, B, 128)
    tail = tail_flat.reshape(NW, NV)
    return _matmul(hist3, w2, tail)
